# Initial kernel scaffold; baseline (speedup 1.0000x reference)
import jax
import jax.numpy as jnp
from jax.experimental import pallas as pl


def kernel(token_id, num_mag, num_pre, num_top, num_low, order, pos_row, pos_col, pos_top, pos_left, format_vec, token_W, mag_W, pre_W, top_W, low_W, order_W, row_W, col_W, tree_W, fmt_W, ln_g, ln_b):
    raise NotImplementedError("write your pallas kernel here")


# speedup vs baseline: 1.0165x; 1.0165x over previous
import functools

import jax
import jax.numpy as jnp
from jax import lax
from jax.experimental import pallas as pl
from jax.experimental.pallas import tpu as pltpu
from jax.experimental.pallas import tpu_sc as plsc

B, S = 4, 2048
N = B * S
H = 768
NUM_EMB = H // 4
UNI_LAYOUT = NUM_EMB // 2
UNI_TREE = (H - NUM_EMB) // 2
EPS = 1e-6

NC, NS, L = 2, 16, 16
NW = NC * NS
PER_W = N // NW
CHUNK = 32
NCHUNK = PER_W // CHUNK
G = H // L


def _sc_partial():
    mesh = plsc.VectorSubcoreMesh(core_axis_name="c", subcore_axis_name="s")

    @functools.partial(
        pl.kernel,
        mesh=mesh,
        out_type=jax.ShapeDtypeStruct((N, H), jnp.float32),
        compiler_params=pltpu.CompilerParams(use_tc_tiling_on_sc=False),
        scratch_types=[
            pltpu.VMEM((CHUNK,), jnp.int32),
            pltpu.VMEM((CHUNK,), jnp.int32),
            pltpu.VMEM((CHUNK,), jnp.int32),
            pltpu.VMEM((CHUNK,), jnp.int32),
            pltpu.VMEM((CHUNK,), jnp.int32),
            pltpu.VMEM((CHUNK,), jnp.int32),
            pltpu.VMEM((CHUNK,), jnp.int32),
            pltpu.VMEM((CHUNK,), jnp.int32),
            pltpu.VMEM((CHUNK, H), jnp.float32),
            pltpu.VMEM((CHUNK, H), jnp.float32),
            pltpu.VMEM((CHUNK, NUM_EMB), jnp.float32),
            pltpu.VMEM((CHUNK, NUM_EMB), jnp.float32),
            pltpu.VMEM((CHUNK, NUM_EMB), jnp.float32),
            pltpu.VMEM((CHUNK, NUM_EMB), jnp.float32),
            pltpu.VMEM((CHUNK, UNI_LAYOUT), jnp.float32),
            pltpu.VMEM((CHUNK, UNI_LAYOUT), jnp.float32),
            pltpu.VMEM((CHUNK, UNI_LAYOUT), jnp.int32),
            pltpu.VMEM((CHUNK, UNI_LAYOUT), jnp.int32),
            pltpu.VMEM((2 * UNI_TREE,), jnp.float32),
            pltpu.SemaphoreType.DMA,
        ],
    )
    def sc_kernel(tok_id, ord_id, mag_id, pre_id, top_id, low_id, row_id,
                  col_id, ptop, pleft, tokW, ordW, magW, preW, topW, lowW,
                  rowW, colW, treeW, out_hbm,
                  i_tok, i_ord, i_mag, i_pre, i_top, i_low, i_row, i_col,
                  x_tok, x_ord, x_mag, x_pre, x_top, x_low, x_row, x_col,
                  p_top, p_left, tw, sem):
        wid = lax.axis_index("s") * NC + lax.axis_index("c")
        pltpu.sync_copy(treeW, tw)

        def chunk_body(c, carry):
            base = wid * PER_W + c * CHUNK
            sl = pl.ds(base, CHUNK)
            pltpu.sync_copy(tok_id.at[sl], i_tok)
            pltpu.sync_copy(ord_id.at[sl], i_ord)
            pltpu.sync_copy(mag_id.at[sl], i_mag)
            pltpu.sync_copy(pre_id.at[sl], i_pre)
            pltpu.sync_copy(top_id.at[sl], i_top)
            pltpu.sync_copy(low_id.at[sl], i_low)
            pltpu.sync_copy(row_id.at[sl], i_row)
            pltpu.sync_copy(col_id.at[sl], i_col)
            pltpu.sync_copy(ptop.at[sl, :], p_top)
            pltpu.sync_copy(pleft.at[sl, :], p_left)
            pltpu.async_copy(tokW.at[i_tok], x_tok, sem).wait()
            pltpu.async_copy(ordW.at[i_ord], x_ord, sem).wait()
            pltpu.async_copy(magW.at[i_mag], x_mag, sem).wait()
            pltpu.async_copy(preW.at[i_pre], x_pre, sem).wait()
            pltpu.async_copy(topW.at[i_top], x_top, sem).wait()
            pltpu.async_copy(lowW.at[i_low], x_low, sem).wait()
            pltpu.async_copy(rowW.at[i_row], x_row, sem).wait()
            pltpu.async_copy(colW.at[i_col], x_col, sem).wait()

            def pos_body(i, carry2):
                ptf = [p_top[i, pl.ds(k * L, L)].astype(jnp.float32)
                       for k in range(6)]
                plf = [p_left[i, pl.ds(k * L, L)].astype(jnp.float32)
                       for k in range(6)]
                num_bufs = (x_mag, x_pre, x_top, x_low)
                for g in range(G):
                    d = pl.ds(g * L, L)
                    x = x_tok[i, d] + x_ord[i, d]
                    x = x + num_bufs[g // 12][i, pl.ds((g % 12) * L, L)]
                    if g < 6:
                        x = x + x_row[i, pl.ds(g * L, L)]
                    elif g < 24:
                        l0 = g * L - UNI_LAYOUT
                        x = x + tw[pl.ds(UNI_TREE + l0, L)] * plf[(l0 // L) % 6]
                    elif g < 30:
                        x = x + x_col[i, pl.ds((g - 24) * L, L)]
                    else:
                        l0 = g * L - 480
                        x = x + tw[pl.ds(l0, L)] * ptf[(l0 // L) % 6]
                    x_tok[i, d] = x
                return carry2

            lax.fori_loop(0, CHUNK, pos_body, 0)
            pltpu.sync_copy(x_tok, out_hbm.at[sl, :])
            return carry

        lax.fori_loop(0, NCHUNK, chunk_body, 0)

    return sc_kernel


_SC_PARTIAL = _sc_partial()

TC_BLK = 512


def _tc_body(part_ref, fv_ref, fmtT_ref, g_ref, b_ref, o_ref):
    x = part_ref[...] + jnp.dot(fv_ref[...], fmtT_ref[...],
                                preferred_element_type=jnp.float32)
    mean = jnp.mean(x, axis=-1, keepdims=True)
    var = jnp.mean((x - mean) ** 2, axis=-1, keepdims=True)
    o_ref[...] = (x - mean) * lax.rsqrt(var + EPS) * g_ref[...] + b_ref[...]


def _tc_finish(partial, fv_pad, fmtT_pad, ln_g, ln_b):
    grid = (N // TC_BLK,)
    return pl.pallas_call(
        _tc_body,
        grid=grid,
        in_specs=[
            pl.BlockSpec((TC_BLK, H), lambda i: (i, 0)),
            pl.BlockSpec((TC_BLK, 16), lambda i: (i, 0)),
            pl.BlockSpec((16, H), lambda i: (0, 0)),
            pl.BlockSpec((H,), lambda i: (0,)),
            pl.BlockSpec((H,), lambda i: (0,)),
        ],
        out_specs=pl.BlockSpec((TC_BLK, H), lambda i: (i, 0)),
        out_shape=jax.ShapeDtypeStruct((N, H), jnp.float32),
    )(partial, fv_pad, fmtT_pad, ln_g, ln_b)


def kernel(token_id, num_mag, num_pre, num_top, num_low, order, pos_row,
           pos_col, pos_top, pos_left, format_vec, token_W, mag_W, pre_W,
           top_W, low_W, order_W, row_W, col_W, tree_W, fmt_W, ln_g, ln_b):
    i32 = jnp.int32
    tok = token_id.reshape(N).astype(i32)
    ordi = order.reshape(N).astype(i32)
    mag = num_mag.reshape(N).astype(i32)
    pre = num_pre.reshape(N).astype(i32)
    top = num_top.reshape(N).astype(i32)
    low = num_low.reshape(N).astype(i32)
    row = pos_row.reshape(N).astype(i32)
    col = pos_col.reshape(N).astype(i32)
    ptop = pos_top.reshape(N, UNI_LAYOUT).astype(i32)
    pleft = pos_left.reshape(N, UNI_LAYOUT).astype(i32)
    treeW_flat = tree_W.reshape(2 * UNI_TREE)

    partial = _SC_PARTIAL(tok, ordi, mag, pre, top, low, row, col,
                          ptop, pleft, token_W, order_W, mag_W, pre_W,
                          top_W, low_W, row_W, col_W, treeW_flat)

    fv_pad = jnp.pad(format_vec.reshape(N, 11), ((0, 0), (0, 5)))
    fmtT_pad = jnp.pad(fmt_W.T, ((0, 5), (0, 0)))
    out = _tc_finish(partial, fv_pad, fmtT_pad, ln_g, ln_b)
    return out.reshape(B, S, H)


# speedup vs baseline: 1.3485x; 1.3265x over previous
import functools

import jax
import jax.numpy as jnp
from jax import lax
from jax.experimental import pallas as pl
from jax.experimental.pallas import tpu as pltpu
from jax.experimental.pallas import tpu_sc as plsc

B, S = 4, 2048
N = B * S
H = 768
NUM_EMB = H // 4
UNI_LAYOUT = NUM_EMB // 2
UNI_TREE = (H - NUM_EMB) // 2
EPS = 1e-6

NC, NS, L = 2, 16, 16
NW = NC * NS
PER_W = N // NW
CHUNK = 16
NCHUNK = PER_W // CHUNK
G = H // L


def _sc_partial():
    mesh = plsc.VectorSubcoreMesh(core_axis_name="c", subcore_axis_name="s")

    buf_set = [
        pltpu.VMEM((CHUNK, H), jnp.float32),
        pltpu.VMEM((CHUNK, H), jnp.float32),
        pltpu.VMEM((4 * CHUNK, NUM_EMB), jnp.float32),
        pltpu.VMEM((2 * CHUNK, UNI_LAYOUT), jnp.float32),
        pltpu.VMEM((CHUNK, UNI_LAYOUT), jnp.int32),
        pltpu.VMEM((CHUNK, UNI_LAYOUT), jnp.int32),
        pltpu.VMEM((CHUNK, H), jnp.float32),
        pltpu.SemaphoreType.DMA,
        pltpu.SemaphoreType.DMA,
    ]

    @functools.partial(
        pl.kernel,
        mesh=mesh,
        out_type=jax.ShapeDtypeStruct((N, H), jnp.float32),
        compiler_params=pltpu.CompilerParams(use_tc_tiling_on_sc=False),
        scratch_types=[
            pltpu.VMEM((PER_W,), jnp.int32),
            pltpu.VMEM((PER_W,), jnp.int32),
            pltpu.VMEM((4 * PER_W,), jnp.int32),
            pltpu.VMEM((2 * PER_W,), jnp.int32),
            pltpu.VMEM((2 * UNI_TREE,), jnp.float32),
        ] + buf_set + buf_set,
    )
    def sc_kernel(tok_id, ord_id, num_id, rc_id, ptop, pleft,
                  tokW, ordW, numW, rcW, treeW, out_hbm,
                  i_tok, i_ord, i_num, i_rc, tw,
                  tok0, ord0, num0, rc0, pt0, pl0, os0, gsem0, ssem0,
                  tok1, ord1, num1, rc1, pt1, pl1, os1, gsem1, ssem1):
        wid = lax.axis_index("s") * NC + lax.axis_index("c")
        w0 = wid * PER_W
        pltpu.sync_copy(treeW, tw)
        pltpu.sync_copy(tok_id.at[pl.ds(w0, PER_W)], i_tok)
        pltpu.sync_copy(ord_id.at[pl.ds(w0, PER_W)], i_ord)
        pltpu.sync_copy(num_id.at[pl.ds(4 * w0, 4 * PER_W)], i_num)
        pltpu.sync_copy(rc_id.at[pl.ds(2 * w0, 2 * PER_W)], i_rc)

        bufs = ((tok0, ord0, num0, rc0, pt0, pl0, os0, gsem0, ssem0),
                (tok1, ord1, num1, rc1, pt1, pl1, os1, gsem1, ssem1))

        def copies(c, bset):
            tokb, ordb, numb, rcb, ptb, plb = bset[:6]
            gsem = bset[7]
            base = w0 + c * CHUNK
            return (
                (tokW.at[i_tok.at[pl.ds(c * CHUNK, CHUNK)]], tokb, gsem),
                (ordW.at[i_ord.at[pl.ds(c * CHUNK, CHUNK)]], ordb, gsem),
                (numW.at[i_num.at[pl.ds(c * 4 * CHUNK, 4 * CHUNK)]], numb,
                 gsem),
                (rcW.at[i_rc.at[pl.ds(c * 2 * CHUNK, 2 * CHUNK)]], rcb, gsem),
                (ptop.at[pl.ds(base, CHUNK), :], ptb, gsem),
                (pleft.at[pl.ds(base, CHUNK), :], plb, gsem),
            )

        def issue(c, bset):
            for src, dst, sem in copies(c, bset):
                pltpu.async_copy(src, dst, sem)

        def drain(c, bset):
            for src, dst, sem in copies(c, bset):
                pltpu.make_async_copy(src, dst, sem).wait()

        issue(0, bufs[0])
        issue(1, bufs[1])

        def chunk_body(c, carry):
            for b in range(2):

                @pl.when(c % 2 == b)
                def _():
                    tokb, ordb, numb, rcb, ptb, plb, osb, gsem, ssem = bufs[b]
                    drain(c, bufs[b])

                    @pl.when(c >= 2)
                    def _():
                        pltpu.make_async_copy(
                            osb, out_hbm.at[pl.ds(w0, CHUNK), :], ssem).wait()

                    def pos_body(i, carry2):
                        ptf = [ptb[i, pl.ds(k * L, L)].astype(jnp.float32)
                               for k in range(6)]
                        plf = [plb[i, pl.ds(k * L, L)].astype(jnp.float32)
                               for k in range(6)]
                        for g in range(G):
                            d = pl.ds(g * L, L)
                            x = tokb[i, d] + ordb[i, d]
                            x = x + numb[4 * i + g // 12,
                                         pl.ds((g % 12) * L, L)]
                            if g < 6:
                                x = x + rcb[2 * i, pl.ds(g * L, L)]
                            elif g < 24:
                                l0 = g * L - UNI_LAYOUT
                                x = x + (tw[pl.ds(UNI_TREE + l0, L)]
                                         * plf[(l0 // L) % 6])
                            elif g < 30:
                                x = x + rcb[2 * i + 1, pl.ds((g - 24) * L, L)]
                            else:
                                l0 = g * L - 480
                                x = x + tw[pl.ds(l0, L)] * ptf[(l0 // L) % 6]
                            osb[i, d] = x
                        return carry2

                    lax.fori_loop(0, CHUNK, pos_body, 0)
                    pltpu.async_copy(
                        osb, out_hbm.at[pl.ds(w0 + c * CHUNK, CHUNK), :],
                        ssem)

                    @pl.when(c < NCHUNK - 2)
                    def _():
                        issue(c + 2, bufs[b])

            return carry

        lax.fori_loop(0, NCHUNK, chunk_body, 0)
        for b in range(2):
            osb, ssem = bufs[b][6], bufs[b][8]
            pltpu.make_async_copy(
                osb, out_hbm.at[pl.ds(w0, CHUNK), :], ssem).wait()

    return sc_kernel


_SC_PARTIAL = _sc_partial()

TC_BLK = 512


def _tc_body(part_ref, fv_ref, fmtT_ref, g_ref, b_ref, o_ref):
    x = part_ref[...] + jnp.dot(fv_ref[...], fmtT_ref[...],
                                preferred_element_type=jnp.float32)
    mean = jnp.mean(x, axis=-1, keepdims=True)
    var = jnp.mean((x - mean) ** 2, axis=-1, keepdims=True)
    o_ref[...] = (x - mean) * lax.rsqrt(var + EPS) * g_ref[...] + b_ref[...]


def _tc_finish(partial, fv_pad, fmtT_pad, ln_g, ln_b):
    grid = (N // TC_BLK,)
    return pl.pallas_call(
        _tc_body,
        grid=grid,
        in_specs=[
            pl.BlockSpec((TC_BLK, H), lambda i: (i, 0)),
            pl.BlockSpec((TC_BLK, 16), lambda i: (i, 0)),
            pl.BlockSpec((16, H), lambda i: (0, 0)),
            pl.BlockSpec((H,), lambda i: (0,)),
            pl.BlockSpec((H,), lambda i: (0,)),
        ],
        out_specs=pl.BlockSpec((TC_BLK, H), lambda i: (i, 0)),
        out_shape=jax.ShapeDtypeStruct((N, H), jnp.float32),
    )(partial, fv_pad, fmtT_pad, ln_g, ln_b)


def kernel(token_id, num_mag, num_pre, num_top, num_low, order, pos_row,
           pos_col, pos_top, pos_left, format_vec, token_W, mag_W, pre_W,
           top_W, low_W, order_W, row_W, col_W, tree_W, fmt_W, ln_g, ln_b):
    i32 = jnp.int32
    tok = token_id.reshape(N).astype(i32)
    ordi = order.reshape(N).astype(i32)
    num_id = jnp.stack(
        [num_mag.reshape(N).astype(i32),
         num_pre.reshape(N).astype(i32) + 12,
         num_top.reshape(N).astype(i32) + 24,
         num_low.reshape(N).astype(i32) + 36], axis=1).reshape(4 * N)
    rc_id = jnp.stack(
        [pos_row.reshape(N).astype(i32),
         pos_col.reshape(N).astype(i32) + 257], axis=1).reshape(2 * N)
    ptop = pos_top.reshape(N, UNI_LAYOUT).astype(i32)
    pleft = pos_left.reshape(N, UNI_LAYOUT).astype(i32)
    numW = jnp.concatenate([mag_W, pre_W, top_W, low_W], axis=0)
    rcW = jnp.concatenate([row_W, col_W], axis=0)
    treeW_flat = tree_W.reshape(2 * UNI_TREE)

    partial = _SC_PARTIAL(tok, ordi, num_id, rc_id, ptop, pleft,
                          token_W, order_W, numW, rcW, treeW_flat)

    fv_pad = jnp.pad(format_vec.reshape(N, 11), ((0, 0), (0, 5)))
    fmtT_pad = jnp.pad(fmt_W.T, ((0, 5), (0, 0)))
    out = _tc_finish(partial, fv_pad, fmtT_pad, ln_g, ln_b)
    return out.reshape(B, S, H)


# speedup vs baseline: 2.0654x; 1.5317x over previous
import functools

import jax
import jax.numpy as jnp
from jax import lax
from jax.experimental import pallas as pl
from jax.experimental.pallas import tpu as pltpu
from jax.experimental.pallas import tpu_sc as plsc

B, S = 4, 2048
N = B * S
H = 768
NUM_EMB = H // 4
UNI_LAYOUT = NUM_EMB // 2
UNI_TREE = (H - NUM_EMB) // 2
EPS = 1e-6

NC, NS, L = 2, 16, 16
NW = NC * NS
PER_W = N // NW
CHUNK = 16
NCHUNK = PER_W // CHUNK
G = H // L


def _sc_partial():
    mesh = plsc.VectorSubcoreMesh(core_axis_name="c", subcore_axis_name="s")

    buf_set = [
        pltpu.VMEM((CHUNK, H), jnp.float32),
        pltpu.VMEM((CHUNK, H), jnp.float32),
        pltpu.VMEM((4 * CHUNK, 256), jnp.float32),
        pltpu.VMEM((2 * CHUNK, 128), jnp.float32),
        pltpu.VMEM((CHUNK, 128), jnp.int32),
        pltpu.VMEM((CHUNK, 128), jnp.int32),
        pltpu.VMEM((CHUNK, H), jnp.float32),
        pltpu.SemaphoreType.DMA,
        pltpu.SemaphoreType.DMA,
    ]

    @functools.partial(
        pl.kernel,
        mesh=mesh,
        out_type=jax.ShapeDtypeStruct((N, H), jnp.float32),
        scratch_types=[
            pltpu.VMEM((PER_W,), jnp.int32),
            pltpu.VMEM((PER_W,), jnp.int32),
            pltpu.VMEM((4 * PER_W,), jnp.int32),
            pltpu.VMEM((2 * PER_W,), jnp.int32),
            pltpu.VMEM((2 * UNI_TREE,), jnp.float32),
        ] + buf_set + buf_set,
    )
    def sc_kernel(tok_id, ord_id, num_id, rc_id, ptop, pleft,
                  tokW, ordW, numW, rcW, treeW, out_hbm,
                  i_tok, i_ord, i_num, i_rc, tw,
                  tok0, ord0, num0, rc0, pt0, pl0, os0, gsem0, ssem0,
                  tok1, ord1, num1, rc1, pt1, pl1, os1, gsem1, ssem1):
        wid = lax.axis_index("s") * NC + lax.axis_index("c")
        w0 = wid * PER_W
        pltpu.sync_copy(treeW, tw)
        pltpu.sync_copy(tok_id.at[pl.ds(w0, PER_W)], i_tok)
        pltpu.sync_copy(ord_id.at[pl.ds(w0, PER_W)], i_ord)
        pltpu.sync_copy(num_id.at[pl.ds(4 * w0, 4 * PER_W)], i_num)
        pltpu.sync_copy(rc_id.at[pl.ds(2 * w0, 2 * PER_W)], i_rc)

        bufs = ((tok0, ord0, num0, rc0, pt0, pl0, os0, gsem0, ssem0),
                (tok1, ord1, num1, rc1, pt1, pl1, os1, gsem1, ssem1))

        def copies(c, bset):
            tokb, ordb, numb, rcb, ptb, plb = bset[:6]
            gsem = bset[7]
            base = w0 + c * CHUNK
            return (
                (tokW.at[i_tok.at[pl.ds(c * CHUNK, CHUNK)]], tokb, gsem),
                (ordW.at[i_ord.at[pl.ds(c * CHUNK, CHUNK)]], ordb, gsem),
                (numW.at[i_num.at[pl.ds(c * 4 * CHUNK, 4 * CHUNK)]], numb,
                 gsem),
                (rcW.at[i_rc.at[pl.ds(c * 2 * CHUNK, 2 * CHUNK)]], rcb, gsem),
                (ptop.at[pl.ds(base, CHUNK), :], ptb, gsem),
                (pleft.at[pl.ds(base, CHUNK), :], plb, gsem),
            )

        def issue(c, bset):
            for src, dst, sem in copies(c, bset):
                pltpu.async_copy(src, dst, sem)

        def drain(c, bset):
            for src, dst, sem in copies(c, bset):
                pltpu.make_async_copy(src, dst, sem).wait()

        issue(0, bufs[0])
        issue(1, bufs[1])

        def chunk_body(c, carry):
            for b in range(2):

                @pl.when(c % 2 == b)
                def _():
                    tokb, ordb, numb, rcb, ptb, plb, osb, gsem, ssem = bufs[b]
                    drain(c, bufs[b])

                    @pl.when(c >= 2)
                    def _():
                        pltpu.make_async_copy(
                            osb, out_hbm.at[pl.ds(w0, CHUNK), :], ssem).wait()

                    def pos_body(i, carry2):
                        ptf = [ptb[i, pl.ds(k * L, L)].astype(jnp.float32)
                               for k in range(6)]
                        plf = [plb[i, pl.ds(k * L, L)].astype(jnp.float32)
                               for k in range(6)]
                        for g in range(G):
                            d = pl.ds(g * L, L)
                            x = tokb[i, d] + ordb[i, d]
                            x = x + numb[4 * i + g // 12,
                                         pl.ds((g % 12) * L, L)]
                            if g < 6:
                                x = x + rcb[2 * i, pl.ds(g * L, L)]
                            elif g < 24:
                                l0 = g * L - UNI_LAYOUT
                                x = x + (tw[pl.ds(UNI_TREE + l0, L)]
                                         * plf[(l0 // L) % 6])
                            elif g < 30:
                                x = x + rcb[2 * i + 1, pl.ds((g - 24) * L, L)]
                            else:
                                l0 = g * L - 480
                                x = x + tw[pl.ds(l0, L)] * ptf[(l0 // L) % 6]
                            osb[i, d] = x
                        return carry2

                    lax.fori_loop(0, CHUNK, pos_body, 0)
                    pltpu.async_copy(
                        osb, out_hbm.at[pl.ds(w0 + c * CHUNK, CHUNK), :],
                        ssem)

                    @pl.when(c < NCHUNK - 2)
                    def _():
                        issue(c + 2, bufs[b])

            return carry

        lax.fori_loop(0, NCHUNK, chunk_body, 0)
        for b in range(2):
            osb, ssem = bufs[b][6], bufs[b][8]
            pltpu.make_async_copy(
                osb, out_hbm.at[pl.ds(w0, CHUNK), :], ssem).wait()

    return sc_kernel


_SC_PARTIAL = _sc_partial()

TC_BLK = 512


def _tc_body(part_ref, fv_ref, fmtT_ref, g_ref, b_ref, o_ref):
    x = part_ref[...] + jnp.dot(fv_ref[...], fmtT_ref[...],
                                preferred_element_type=jnp.float32)
    mean = jnp.mean(x, axis=-1, keepdims=True)
    var = jnp.mean((x - mean) ** 2, axis=-1, keepdims=True)
    o_ref[...] = (x - mean) * lax.rsqrt(var + EPS) * g_ref[...] + b_ref[...]


def _tc_finish(partial, fv_pad, fmtT_pad, ln_g, ln_b):
    grid = (N // TC_BLK,)
    return pl.pallas_call(
        _tc_body,
        grid=grid,
        in_specs=[
            pl.BlockSpec((TC_BLK, H), lambda i: (i, 0)),
            pl.BlockSpec((TC_BLK, 16), lambda i: (i, 0)),
            pl.BlockSpec((16, H), lambda i: (0, 0)),
            pl.BlockSpec((H,), lambda i: (0,)),
            pl.BlockSpec((H,), lambda i: (0,)),
        ],
        out_specs=pl.BlockSpec((TC_BLK, H), lambda i: (i, 0)),
        out_shape=jax.ShapeDtypeStruct((N, H), jnp.float32),
    )(partial, fv_pad, fmtT_pad, ln_g, ln_b)


def kernel(token_id, num_mag, num_pre, num_top, num_low, order, pos_row,
           pos_col, pos_top, pos_left, format_vec, token_W, mag_W, pre_W,
           top_W, low_W, order_W, row_W, col_W, tree_W, fmt_W, ln_g, ln_b):
    i32 = jnp.int32
    tok = token_id.reshape(N).astype(i32)
    ordi = order.reshape(N).astype(i32)
    num_id = jnp.stack(
        [num_mag.reshape(N).astype(i32),
         num_pre.reshape(N).astype(i32) + 12,
         num_top.reshape(N).astype(i32) + 24,
         num_low.reshape(N).astype(i32) + 36], axis=1).reshape(4 * N)
    rc_id = jnp.stack(
        [pos_row.reshape(N).astype(i32),
         pos_col.reshape(N).astype(i32) + 257], axis=1).reshape(2 * N)
    ptop = jnp.pad(pos_top.reshape(N, UNI_LAYOUT).astype(i32),
                   ((0, 0), (0, 128 - UNI_LAYOUT)))
    pleft = jnp.pad(pos_left.reshape(N, UNI_LAYOUT).astype(i32),
                    ((0, 0), (0, 128 - UNI_LAYOUT)))
    numW = jnp.pad(jnp.concatenate([mag_W, pre_W, top_W, low_W], axis=0),
                   ((0, 0), (0, 256 - NUM_EMB)))
    rcW = jnp.pad(jnp.concatenate([row_W, col_W], axis=0),
                  ((0, 0), (0, 128 - UNI_LAYOUT)))
    treeW_flat = tree_W.reshape(2 * UNI_TREE)

    partial = _SC_PARTIAL(tok, ordi, num_id, rc_id, ptop, pleft,
                          token_W, order_W, numW, rcW, treeW_flat)

    fv_pad = jnp.pad(format_vec.reshape(N, 11), ((0, 0), (0, 5)))
    fmtT_pad = jnp.pad(fmt_W.T, ((0, 5), (0, 0)))
    out = _tc_finish(partial, fv_pad, fmtT_pad, ln_g, ln_b)
    return out.reshape(B, S, H)


# speedup vs baseline: 2.5173x; 1.2188x over previous
import functools

import jax
import jax.numpy as jnp
from jax import lax
from jax.experimental import pallas as pl
from jax.experimental.pallas import tpu as pltpu
from jax.experimental.pallas import tpu_sc as plsc

B, S = 4, 2048
N = B * S
H = 768
NUM_EMB = H // 4
UNI_LAYOUT = NUM_EMB // 2
UNI_TREE = (H - NUM_EMB) // 2
EPS = 1e-6

NC, NS, L = 2, 16, 16
NW = NC * NS
PER_W = N // NW
CHUNK = 8
NCHUNK = PER_W // CHUNK
G = H // L

OFF_MAG = 0
OFF_PRE = OFF_MAG + 12 * NUM_EMB
OFF_TOP = OFF_PRE + 12 * NUM_EMB
OFF_LOW = OFF_TOP + 12 * NUM_EMB
OFF_ROW = OFF_LOW + 12 * NUM_EMB
OFF_COL = OFF_ROW + 257 * UNI_LAYOUT
OFF_TREE = OFF_COL + 257 * UNI_LAYOUT
TABS_LEN = OFF_TREE + 2 * UNI_TREE


def _sc_partial():
    mesh = plsc.VectorSubcoreMesh(core_axis_name="c", subcore_axis_name="s")

    buf_set = [
        pltpu.VMEM((CHUNK, H), jnp.float32),
        pltpu.VMEM((CHUNK, H), jnp.float32),
        pltpu.VMEM((CHUNK, UNI_LAYOUT), jnp.int32),
        pltpu.VMEM((CHUNK, UNI_LAYOUT), jnp.int32),
        pltpu.VMEM((CHUNK, H), jnp.float32),
        pltpu.SemaphoreType.DMA,
        pltpu.SemaphoreType.DMA,
    ]

    @functools.partial(
        pl.kernel,
        mesh=mesh,
        out_type=jax.ShapeDtypeStruct((N, H), jnp.float32),
        compiler_params=pltpu.CompilerParams(needs_layout_passes=False),
        scratch_types=[
            pltpu.VMEM((PER_W,), jnp.int32),
            pltpu.VMEM((PER_W,), jnp.int32),
            pltpu.VMEM((PER_W,), jnp.int32),
            pltpu.VMEM((PER_W,), jnp.int32),
            pltpu.VMEM((PER_W,), jnp.int32),
            pltpu.VMEM((PER_W,), jnp.int32),
            pltpu.VMEM((PER_W,), jnp.int32),
            pltpu.VMEM((PER_W,), jnp.int32),
            pltpu.VMEM((TABS_LEN,), jnp.float32),
        ] + buf_set + buf_set,
    )
    def sc_kernel(tok_id, ord_id, mag_id, pre_id, top_id, low_id, row_id,
                  col_id, ptop, pleft, tokW, ordW, tabs, out_hbm,
                  i_tok, i_ord, i_mag, i_pre, i_top, i_low, i_row, i_col,
                  tv,
                  tok0, ord0, pt0, pl0, os0, gsem0, ssem0,
                  tok1, ord1, pt1, pl1, os1, gsem1, ssem1):
        wid = lax.axis_index("s") * NC + lax.axis_index("c")
        w0 = wid * PER_W
        br = w0 // S
        col0 = w0 % S
        pltpu.sync_copy(tabs, tv)
        pltpu.sync_copy(tok_id.at[br, pl.ds(col0, PER_W)], i_tok)
        pltpu.sync_copy(ord_id.at[br, pl.ds(col0, PER_W)], i_ord)
        pltpu.sync_copy(mag_id.at[br, pl.ds(col0, PER_W)], i_mag)
        pltpu.sync_copy(pre_id.at[br, pl.ds(col0, PER_W)], i_pre)
        pltpu.sync_copy(top_id.at[br, pl.ds(col0, PER_W)], i_top)
        pltpu.sync_copy(low_id.at[br, pl.ds(col0, PER_W)], i_low)
        pltpu.sync_copy(row_id.at[br, pl.ds(col0, PER_W)], i_row)
        pltpu.sync_copy(col_id.at[br, pl.ds(col0, PER_W)], i_col)

        bufs = ((tok0, ord0, pt0, pl0, os0, gsem0, ssem0),
                (tok1, ord1, pt1, pl1, os1, gsem1, ssem1))

        def copies(c, bset):
            tokb, ordb, ptb, plb = bset[:4]
            gsem = bset[5]
            cc = col0 + c * CHUNK
            return (
                (tokW.at[i_tok.at[pl.ds(c * CHUNK, CHUNK)]], tokb, gsem),
                (ordW.at[i_ord.at[pl.ds(c * CHUNK, CHUNK)]], ordb, gsem),
                (ptop.at[br, pl.ds(cc, CHUNK), :], ptb, gsem),
                (pleft.at[br, pl.ds(cc, CHUNK), :], plb, gsem),
            )

        def issue(c, bset):
            for src, dst, sem in copies(c, bset):
                pltpu.async_copy(src, dst, sem)

        def drain(c, bset):
            for src, dst, sem in copies(c, bset):
                pltpu.make_async_copy(src, dst, sem).wait()

        issue(0, bufs[0])
        issue(1, bufs[1])

        iota = jnp.arange(L, dtype=jnp.int32)

        def chunk_body(c, carry):
            for b in range(2):

                @pl.when(c % 2 == b)
                def _():
                    tokb, ordb, ptb, plb, osb, gsem, ssem = bufs[b]
                    drain(c, bufs[b])

                    @pl.when(c >= 2)
                    def _():
                        pltpu.make_async_copy(
                            osb, out_hbm.at[pl.ds(w0, CHUNK), :], ssem).wait()

                    def pos_body(i, carry2):
                        p = c * CHUNK + i
                        pv = jnp.full((L,), p, jnp.int32)
                        m_mag = plsc.load_gather(i_mag, [pv])
                        m_pre = plsc.load_gather(i_pre, [pv])
                        m_top = plsc.load_gather(i_top, [pv])
                        m_low = plsc.load_gather(i_low, [pv])
                        m_row = plsc.load_gather(i_row, [pv])
                        m_col = plsc.load_gather(i_col, [pv])
                        bases = (m_mag * NUM_EMB + iota + OFF_MAG,
                                 m_pre * NUM_EMB + iota + OFF_PRE,
                                 m_top * NUM_EMB + iota + OFF_TOP,
                                 m_low * NUM_EMB + iota + OFF_LOW)
                        b_row = m_row * UNI_LAYOUT + iota + OFF_ROW
                        b_col = m_col * UNI_LAYOUT + iota + OFF_COL
                        ptf = [ptb[i, pl.ds(k * L, L)].astype(jnp.float32)
                               for k in range(6)]
                        plf = [plb[i, pl.ds(k * L, L)].astype(jnp.float32)
                               for k in range(6)]
                        for g in range(G):
                            d = pl.ds(g * L, L)
                            x = tokb[i, d] + ordb[i, d]
                            x = x + plsc.load_gather(
                                tv, [bases[g // 12] + (g % 12) * L])
                            if g < 6:
                                x = x + plsc.load_gather(tv, [b_row + g * L])
                            elif g < 24:
                                l0 = g * L - UNI_LAYOUT
                                x = x + (tv[pl.ds(OFF_TREE + UNI_TREE + l0, L)]
                                         * plf[(l0 // L) % 6])
                            elif g < 30:
                                x = x + plsc.load_gather(
                                    tv, [b_col + (g - 24) * L])
                            else:
                                l0 = g * L - 480
                                x = x + (tv[pl.ds(OFF_TREE + l0, L)]
                                         * ptf[(l0 // L) % 6])
                            osb[i, d] = x
                        return carry2

                    lax.fori_loop(0, CHUNK, pos_body, 0)
                    pltpu.async_copy(
                        osb, out_hbm.at[pl.ds(w0 + c * CHUNK, CHUNK), :],
                        ssem)

                    @pl.when(c < NCHUNK - 2)
                    def _():
                        issue(c + 2, bufs[b])

            return carry

        lax.fori_loop(0, NCHUNK, chunk_body, 0)
        for b in range(2):
            osb, ssem = bufs[b][4], bufs[b][6]
            pltpu.make_async_copy(
                osb, out_hbm.at[pl.ds(w0, CHUNK), :], ssem).wait()

    return sc_kernel


_SC_PARTIAL = _sc_partial()

TC_BLK = 512


def _tc_body(part_ref, fv_ref, fmtT_ref, g_ref, b_ref, o_ref):
    x = part_ref[...] + jnp.dot(fv_ref[...], fmtT_ref[...],
                                preferred_element_type=jnp.float32)
    mean = jnp.mean(x, axis=-1, keepdims=True)
    var = jnp.mean((x - mean) ** 2, axis=-1, keepdims=True)
    o_ref[...] = (x - mean) * lax.rsqrt(var + EPS) * g_ref[...] + b_ref[...]


def _tc_finish(partial, fv_pad, fmtT_pad, ln_g, ln_b):
    grid = (N // TC_BLK,)
    return pl.pallas_call(
        _tc_body,
        grid=grid,
        in_specs=[
            pl.BlockSpec((TC_BLK, H), lambda i: (i, 0)),
            pl.BlockSpec((TC_BLK, 16), lambda i: (i, 0)),
            pl.BlockSpec((16, H), lambda i: (0, 0)),
            pl.BlockSpec((H,), lambda i: (0,)),
            pl.BlockSpec((H,), lambda i: (0,)),
        ],
        out_specs=pl.BlockSpec((TC_BLK, H), lambda i: (i, 0)),
        out_shape=jax.ShapeDtypeStruct((N, H), jnp.float32),
    )(partial, fv_pad, fmtT_pad, ln_g, ln_b)


def kernel(token_id, num_mag, num_pre, num_top, num_low, order, pos_row,
           pos_col, pos_top, pos_left, format_vec, token_W, mag_W, pre_W,
           top_W, low_W, order_W, row_W, col_W, tree_W, fmt_W, ln_g, ln_b):
    i32 = jnp.int32
    tabs = jnp.concatenate([
        mag_W.ravel(), pre_W.ravel(), top_W.ravel(), low_W.ravel(),
        row_W.ravel(), col_W.ravel(), tree_W.ravel()])

    partial = _SC_PARTIAL(
        token_id.astype(i32), order.astype(i32), num_mag.astype(i32),
        num_pre.astype(i32), num_top.astype(i32), num_low.astype(i32),
        pos_row.astype(i32), pos_col.astype(i32),
        pos_top.astype(i32), pos_left.astype(i32),
        token_W, order_W, tabs)

    fv_pad = jnp.pad(format_vec.reshape(N, 11), ((0, 0), (0, 5)))
    fmtT_pad = jnp.pad(fmt_W.T, ((0, 5), (0, 0)))
    out = _tc_finish(partial, fv_pad, fmtT_pad, ln_g, ln_b)
    return out.reshape(B, S, H)
